# baseline (speedup 1.0000x reference)
import functools

import numpy as np

import jax
import jax.numpy as jnp
from jax import lax
from jax.experimental import pallas as pl
from jax.experimental.pallas import tpu as pltpu
from jax.experimental.pallas import tpu_sc as plsc

_N = 10000
_E = 320000
_D = 128

_NC = 2
_NS = 16
_NW = _NC * _NS
_L = 16
_PPW = (_D // 2) // _NW
_CHUNK = 3200
_GROUPS = _CHUNK // _L
_BLK = 4
_NBLKS = _GROUPS // _BLK
_NCHUNKS = _E // _CHUNK

_NEGINF_PAIR = -8323200


def _unpack(w):
  lo = plsc.bitcast(jnp.left_shift(w, 16), jnp.float32)
  hi = plsc.bitcast(w & jnp.int32(-65536), jnp.float32)
  return lo, hi


def _pack(lo, hi):
  return plsc.bitcast(hi, jnp.int32) | lax.shift_right_logical(
      plsc.bitcast(lo, jnp.int32), 16)


def _sc_segmax(qp, src, dst):
  mesh = plsc.VectorSubcoreMesh(core_axis_name="c", subcore_axis_name="s")

  @functools.partial(
      pl.kernel,
      out_type=jax.ShapeDtypeStruct((_D * _N,), jnp.float32),
      mesh=mesh,
      compiler_params=pltpu.CompilerParams(needs_layout_passes=False),
      scratch_types=[
          pltpu.VMEM((_PPW * _N,), jnp.int32),
          pltpu.VMEM((_PPW * _N,), jnp.int32),
          pltpu.VMEM((_PPW * _N,), jnp.int32),
          pltpu.VMEM((2 * _N,), jnp.float32),
          pltpu.VMEM((2 * _N,), jnp.float32),
          pltpu.VMEM((_CHUNK,), jnp.int32),
          pltpu.VMEM((_CHUNK,), jnp.int32),
          pltpu.VMEM((_CHUNK,), jnp.int32),
          pltpu.VMEM((_CHUNK,), jnp.int32),
          pltpu.SemaphoreType.DMA,
          pltpu.SemaphoreType.DMA,
          pltpu.SemaphoreType.DMA,
          pltpu.SemaphoreType.DMA,
      ],
  )
  def k(qp_hbm, src_hbm, dst_hbm, out_hbm, q_v, acce_v, acco_v, slo_v, shi_v,
        sa_v, da_v, sb_v, db_v, sem_sa, sem_da, sem_sb, sem_db):
    wid = lax.axis_index("s") * _NC + lax.axis_index("c")
    pbase = wid * _PPW * _N
    pltpu.sync_copy(qp_hbm.at[pl.ds(pbase, _PPW * _N)], q_v)

    negw = jnp.full((_L,), _NEGINF_PAIR, dtype=jnp.int32)
    poff = [jnp.full((_L,), p * _N, dtype=jnp.int32) for p in range(_PPW)]

    def init_row(i, _):
      acce_v[pl.ds(i * _L, _L)] = negw
      acco_v[pl.ds(i * _L, _L)] = negw
      return 0

    lax.fori_loop(0, (_PPW * _N) // _L, init_row, 0)

    ones = jnp.ones((_L,), dtype=jnp.int32)
    lid = jnp.arange(_L, dtype=jnp.int32)
    idxp1 = jnp.minimum(lid + 1, _L - 1)
    lt15 = lid < (_L - 1)

    def slow_group(acc_v, s_v, d_v, g):
      s16 = s_v[pl.ds(g * _L, _L)]
      d16 = d_v[pl.ds(g * _L, _L)]
      qw = [plsc.load_gather(q_v, [s16 + poff[p]]) for p in range(_PPW)]
      vals = [_unpack(w) for w in qw]
      didx = [d16 + poff[p] for p in range(_PPW)]

      def cond(pend):
        n = jnp.sum(pend[0])
        for p in range(1, _PPW):
          n = n + jnp.sum(pend[p])
        return n > 0

      def attempt(pend):
        for p in range(_PPW):
          cur = plsc.load_gather(acc_v, [didx[p]])
          clo, chi = _unpack(cur)
          neww = _pack(jnp.maximum(clo, vals[p][0]),
                       jnp.maximum(chi, vals[p][1]))
          plsc.store_scatter(acc_v, [didx[p]], neww, mask=pend[p] > 0)
        out = []
        for p in range(_PPW):
          chk = plsc.load_gather(acc_v, [didx[p]])
          klo, khi = _unpack(chk)
          bad = (klo < vals[p][0]) | (khi < vals[p][1])
          out.append(pend[p] & bad.astype(jnp.int32))
        return tuple(out)

      lax.while_loop(cond, attempt, (ones,) * _PPW)

    def block(s_v, d_v, bi, _):
      g0 = bi * _BLK
      fails = None
      for t in range(_BLK):
        g = g0 + t
        acc_v = acce_v
        s16 = s_v[pl.ds(g * _L, _L)]
        d16 = d_v[pl.ds(g * _L, _L)]
        qw = [plsc.load_gather(q_v, [s16 + poff[p]]) for p in range(_PPW)]
        vals = [_unpack(w) for w in qw]
        didx = [d16 + poff[p] for p in range(_PPW)]
        ds = lax.sort(d16)
        nxt = lax.gather(
            ds, idxp1.reshape(_L, 1),
            lax.GatherDimensionNumbers(offset_dims=(), collapsed_slice_dims=(0,),
                                       start_index_map=(0,)),
            (1,), mode=lax.GatherScatterMode.PROMISE_IN_BOUNDS)
        f = ((ds == nxt) & lt15).astype(jnp.int32)
        fails = f if fails is None else (fails | f)
        for p in range(_PPW):
          cur = plsc.load_gather(acc_v, [didx[p]])
          clo, chi = _unpack(cur)
          neww = _pack(jnp.maximum(clo, vals[p][0]),
                       jnp.maximum(chi, vals[p][1]))
          plsc.store_scatter(acc_v, [didx[p]], neww)

      nfail = plsc.all_reduce_population_count(fails > 0)[0]

      @pl.when(nfail != 0)
      def _slow():
        for t in range(_BLK):
          acc_v = acce_v if t % 2 == 0 else acco_v
          slow_group(acc_v, s_v, d_v, g0 + t)

      return 0

    def do_chunk(s_v, d_v):
      lax.fori_loop(0, _NBLKS, functools.partial(block, s_v, d_v), 0,
                    unroll=2)

    def start_copy(c, s_v, d_v, sem_s, sem_d):
      off = c * _CHUNK
      pltpu.async_copy(src_hbm.at[pl.ds(off, _CHUNK)], s_v, sem_s)
      pltpu.async_copy(dst_hbm.at[pl.ds(off, _CHUNK)], d_v, sem_d)

    def wait_copy(s_v, d_v, sem_s, sem_d):
      pltpu.make_async_copy(src_hbm.at[pl.ds(0, _CHUNK)], s_v, sem_s).wait()
      pltpu.make_async_copy(dst_hbm.at[pl.ds(0, _CHUNK)], d_v, sem_d).wait()

    start_copy(0, sa_v, da_v, sem_sa, sem_da)

    def chunk_pair(c2, _):
      c = c2 * 2
      start_copy(c + 1, sb_v, db_v, sem_sb, sem_db)
      wait_copy(sa_v, da_v, sem_sa, sem_da)
      do_chunk(sa_v, da_v)

      @pl.when(c2 + 1 < _NCHUNKS // 2)
      def _prefetch():
        start_copy(c + 2, sa_v, da_v, sem_sa, sem_da)

      wait_copy(sb_v, db_v, sem_sb, sem_db)
      do_chunk(sb_v, db_v)
      return 0

    lax.fori_loop(0, _NCHUNKS // 2, chunk_pair, 0)

    def merge_row(i, _):
      sl = pl.ds(i * _L, _L)
      elo, ehi = _unpack(acce_v[sl])
      olo, ohi = _unpack(acco_v[sl])
      slo_v[sl] = jnp.maximum(elo, olo)
      shi_v[sl] = jnp.maximum(ehi, ohi)
      return 0

    lax.fori_loop(0, (_PPW * _N) // _L, merge_row, 0, unroll=2)
    pltpu.sync_copy(slo_v, out_hbm.at[pl.ds(2 * wid * _N, _PPW * _N)])
    pltpu.sync_copy(shi_v, out_hbm.at[pl.ds((64 + 2 * wid) * _N, _PPW * _N)])

  return k(qp.reshape((_D // 2) * _N), src, dst).reshape(_D, _N)


def _pack_q_tc(q):
  lo = lax.bitcast_convert_type(q[:_D // 2].astype(jnp.bfloat16),
                                jnp.uint16).astype(jnp.uint32)
  hi = lax.bitcast_convert_type(q[_D // 2:].astype(jnp.bfloat16),
                                jnp.uint16).astype(jnp.uint32)
  return lax.bitcast_convert_type((hi << 16) | lo, jnp.int32)


def _tc1_body(x_ref, w_ref, b_ref, p_ref, q_ref):
  x = x_ref[...]
  w = w_ref[...]
  wd = w[:_D] - w[_D:]
  ws = w[_D:]
  dn = (((0,), (1,)), ((), ()))
  p_ref[...] = lax.dot_general(wd, x, dn, preferred_element_type=jnp.float32) + b_ref[...]
  q_ref[...] = _pack_q_tc(
      lax.dot_general(ws, x, dn, preferred_element_type=jnp.float32))


def _tc2_body(p_ref, s_ref, w_ref, b_ref, p2_ref, q2_ref):
  s = s_ref[...]
  h = jnp.where(s > -jnp.inf, p_ref[...] + s, 0.0)
  h = jnp.maximum(h, 0.0)
  w = w_ref[...]
  wd = w[:_D] - w[_D:]
  ws = w[_D:]
  dn = (((0,), (0,)), ((), ()))
  p2_ref[...] = lax.dot_general(wd, h, dn, preferred_element_type=jnp.float32) + b_ref[...]
  q2_ref[...] = _pack_q_tc(
      lax.dot_general(ws, h, dn, preferred_element_type=jnp.float32))


def _tc3_body(p_ref, s_ref, pm_ref, o_ref):
  s = s_ref[...]
  o = jnp.where(s > -jnp.inf, p_ref[...] + s, 0.0)
  m = jnp.max(o, axis=0, keepdims=True)
  lse = jnp.log(jnp.sum(jnp.exp(o - m), axis=0, keepdims=True)) + m
  lsm = o - lse
  o_ref[...] = jnp.dot(pm_ref[...], lsm, preferred_element_type=jnp.float32).T


_PERM = np.concatenate([np.arange(0, _D, 2), np.arange(1, _D, 2)])
_PMAT = np.zeros((_D, _D), dtype=np.float32)
_PMAT[_PERM, np.arange(_D)] = 1.0


def kernel(x, edge_index, W1, b1, W2, b2):
  src = edge_index[0]
  dst = edge_index[1]
  perm = jnp.asarray(_PERM)
  w1p = W1[:, perm]
  b1c = b1[perm].reshape(_D, 1)
  w2p = W2[jnp.concatenate([perm, _D + perm])][:, perm]
  b2c = b2[perm].reshape(_D, 1)
  pmat = jnp.asarray(_PMAT)

  ft = jax.ShapeDtypeStruct((_D, _N), jnp.float32)
  qt = jax.ShapeDtypeStruct((_D // 2, _N), jnp.int32)

  p1t, q1p = pl.pallas_call(
      _tc1_body,
      out_shape=[ft, qt],
  )(x, w1p, b1c)

  s1t = _sc_segmax(q1p, src, dst)

  p2t, q2p = pl.pallas_call(
      _tc2_body,
      out_shape=[ft, qt],
  )(p1t, s1t, w2p, b2c)

  s2t = _sc_segmax(q2p, src, dst)

  out = pl.pallas_call(
      _tc3_body,
      out_shape=jax.ShapeDtypeStruct((_N, _D), jnp.float32),
  )(p2t, s2t, pmat)

  return out


# speedup vs baseline: 1.3230x; 1.3230x over previous
import functools

import numpy as np

import jax
import jax.numpy as jnp
from jax import lax
from jax.experimental import pallas as pl
from jax.experimental.pallas import tpu as pltpu
from jax.experimental.pallas import tpu_sc as plsc

_N = 10000
_E = 320000
_D = 128
_W = _D // 2

_NC = 2
_NS = 16
_NW = _NC * _NS
_L = 16
_PPT = 4
_EH = _E // 2
_CHUNK = 3200
_GROUPS = _CHUNK // _L
_BLK = 4
_NBLKS = _GROUPS // _BLK
_NCHUNKS = _EH // _CHUNK

_NEGINF_PAIR = -8323200


def _unpack(w):
  lo = plsc.bitcast(jnp.left_shift(w, 16), jnp.float32)
  hi = plsc.bitcast(w & jnp.int32(-65536), jnp.float32)
  return lo, hi


def _pack(lo, hi):
  return plsc.bitcast(hi, jnp.int32) | lax.shift_right_logical(
      plsc.bitcast(lo, jnp.int32), 16)


def _sc_segmax(qp, src, dst):
  mesh = plsc.VectorSubcoreMesh(core_axis_name="c", subcore_axis_name="s")

  @functools.partial(
      pl.kernel,
      out_type=jax.ShapeDtypeStruct((2 * _W * _N,), jnp.int32),
      mesh=mesh,
      compiler_params=pltpu.CompilerParams(needs_layout_passes=False),
      scratch_types=[
          pltpu.VMEM((_PPT * _N,), jnp.int32),
          pltpu.VMEM((_PPT * _N,), jnp.int32),
          pltpu.VMEM((_CHUNK,), jnp.int32),
          pltpu.VMEM((_CHUNK,), jnp.int32),
          pltpu.VMEM((_CHUNK,), jnp.int32),
          pltpu.VMEM((_CHUNK,), jnp.int32),
          pltpu.SemaphoreType.DMA,
          pltpu.SemaphoreType.DMA,
          pltpu.SemaphoreType.DMA,
          pltpu.SemaphoreType.DMA,
      ],
  )
  def k(qp_hbm, src_hbm, dst_hbm, out_hbm, q_v, acc_v,
        sa_v, da_v, sb_v, db_v, sem_sa, sem_da, sem_sb, sem_db):
    wid = lax.axis_index("s") * _NC + lax.axis_index("c")
    half = wid // _NS
    slot = wid % _NS
    ebase = half * _EH
    pltpu.sync_copy(qp_hbm.at[pl.ds(slot * _PPT * _N, _PPT * _N)], q_v)

    negw = jnp.full((_L,), _NEGINF_PAIR, dtype=jnp.int32)
    poff = [jnp.full((_L,), p * _N, dtype=jnp.int32) for p in range(_PPT)]

    def init_row(i, _):
      acc_v[pl.ds(i * _L, _L)] = negw
      return 0

    lax.fori_loop(0, (_PPT * _N) // _L, init_row, 0)

    ones = jnp.ones((_L,), dtype=jnp.int32)
    lid = jnp.arange(_L, dtype=jnp.int32)
    idxp1 = jnp.minimum(lid + 1, _L - 1)
    lt15 = lid < (_L - 1)

    def slow_group(s_v, d_v, g):
      s16 = s_v[pl.ds(g * _L, _L)]
      d16 = d_v[pl.ds(g * _L, _L)]
      qw = [plsc.load_gather(q_v, [s16 + poff[p]]) for p in range(_PPT)]
      vals = [_unpack(w) for w in qw]
      didx = [d16 + poff[p] for p in range(_PPT)]

      def cond(pend):
        n = jnp.sum(pend[0])
        for p in range(1, _PPT):
          n = n + jnp.sum(pend[p])
        return n > 0

      def attempt(pend):
        for p in range(_PPT):
          cur = plsc.load_gather(acc_v, [didx[p]])
          clo, chi = _unpack(cur)
          neww = _pack(jnp.maximum(clo, vals[p][0]),
                       jnp.maximum(chi, vals[p][1]))
          plsc.store_scatter(acc_v, [didx[p]], neww, mask=pend[p] > 0)
        out = []
        for p in range(_PPT):
          chk = plsc.load_gather(acc_v, [didx[p]])
          klo, khi = _unpack(chk)
          bad = (klo < vals[p][0]) | (khi < vals[p][1])
          out.append(pend[p] & bad.astype(jnp.int32))
        return tuple(out)

      lax.while_loop(cond, attempt, (ones,) * _PPT)

    def block(s_v, d_v, bi, _):
      g0 = bi * _BLK
      fails = None
      for t in range(_BLK):
        g = g0 + t
        s16 = s_v[pl.ds(g * _L, _L)]
        d16 = d_v[pl.ds(g * _L, _L)]
        qw = [plsc.load_gather(q_v, [s16 + poff[p]]) for p in range(_PPT)]
        vals = [_unpack(w) for w in qw]
        didx = [d16 + poff[p] for p in range(_PPT)]
        ds = lax.sort(d16)
        nxt = lax.gather(
            ds, idxp1.reshape(_L, 1),
            lax.GatherDimensionNumbers(offset_dims=(), collapsed_slice_dims=(0,),
                                       start_index_map=(0,)),
            (1,), mode=lax.GatherScatterMode.PROMISE_IN_BOUNDS)
        f = ((ds == nxt) & lt15).astype(jnp.int32)
        fails = f if fails is None else (fails | f)
        for p in range(_PPT):
          cur = plsc.load_gather(acc_v, [didx[p]])
          clo, chi = _unpack(cur)
          neww = _pack(jnp.maximum(clo, vals[p][0]),
                       jnp.maximum(chi, vals[p][1]))
          plsc.store_scatter(acc_v, [didx[p]], neww)

      nfail = plsc.all_reduce_population_count(fails > 0)[0]

      @pl.when(nfail != 0)
      def _slow():
        for t in range(_BLK):
          slow_group(s_v, d_v, g0 + t)

      return 0

    def do_chunk(s_v, d_v):
      lax.fori_loop(0, _NBLKS, functools.partial(block, s_v, d_v), 0,
                    unroll=2)

    def start_copy(c, s_v, d_v, sem_s, sem_d):
      off = ebase + c * _CHUNK
      pltpu.async_copy(src_hbm.at[pl.ds(off, _CHUNK)], s_v, sem_s)
      pltpu.async_copy(dst_hbm.at[pl.ds(off, _CHUNK)], d_v, sem_d)

    def wait_copy(s_v, d_v, sem_s, sem_d):
      pltpu.make_async_copy(src_hbm.at[pl.ds(0, _CHUNK)], s_v, sem_s).wait()
      pltpu.make_async_copy(dst_hbm.at[pl.ds(0, _CHUNK)], d_v, sem_d).wait()

    start_copy(0, sa_v, da_v, sem_sa, sem_da)

    def chunk_pair(c2, _):
      c = c2 * 2
      start_copy(c + 1, sb_v, db_v, sem_sb, sem_db)
      wait_copy(sa_v, da_v, sem_sa, sem_da)
      do_chunk(sa_v, da_v)

      @pl.when(c2 + 1 < _NCHUNKS // 2)
      def _prefetch():
        start_copy(c + 2, sa_v, da_v, sem_sa, sem_da)

      wait_copy(sb_v, db_v, sem_sb, sem_db)
      do_chunk(sb_v, db_v)
      return 0

    lax.fori_loop(0, _NCHUNKS // 2, chunk_pair, 0)

    pltpu.sync_copy(
        acc_v, out_hbm.at[pl.ds((half * _W + slot * _PPT) * _N, _PPT * _N)])

  return k(qp.reshape(_W * _N), src, dst).reshape(2, _W, _N)


def _pack_q_tc(q):
  lo = lax.bitcast_convert_type(q[:_W].astype(jnp.bfloat16),
                                jnp.uint16).astype(jnp.uint32)
  hi = lax.bitcast_convert_type(q[_W:].astype(jnp.bfloat16),
                                jnp.uint16).astype(jnp.uint32)
  return lax.bitcast_convert_type((hi << 16) | lo, jnp.int32)


def _merge_s_tc(s2):
  out = None
  for h in range(2):
    w = s2[h]
    lo = lax.bitcast_convert_type(jnp.left_shift(w, 16), jnp.float32)
    hi = lax.bitcast_convert_type(w & jnp.int32(-65536), jnp.float32)
    s = jnp.concatenate([lo, hi], axis=0)
    out = s if out is None else jnp.maximum(out, s)
  return out


def _tc1_body(x_ref, w_ref, b_ref, p_ref, q_ref):
  x = x_ref[...]
  w = w_ref[...]
  wd = w[:_D] - w[_D:]
  ws = w[_D:]
  dn = (((0,), (1,)), ((), ()))
  p_ref[...] = lax.dot_general(wd, x, dn, preferred_element_type=jnp.float32) + b_ref[...]
  q_ref[...] = _pack_q_tc(
      lax.dot_general(ws, x, dn, preferred_element_type=jnp.float32))


def _tc2_body(p_ref, s_ref, w_ref, b_ref, p2_ref, q2_ref):
  s = _merge_s_tc(s_ref[...])
  h = jnp.where(s > -jnp.inf, p_ref[...] + s, 0.0)
  h = jnp.maximum(h, 0.0)
  w = w_ref[...]
  wd = w[:_D] - w[_D:]
  ws = w[_D:]
  dn = (((0,), (0,)), ((), ()))
  p2_ref[...] = lax.dot_general(wd, h, dn, preferred_element_type=jnp.float32) + b_ref[...]
  q2_ref[...] = _pack_q_tc(
      lax.dot_general(ws, h, dn, preferred_element_type=jnp.float32))


def _tc3_body(p_ref, s_ref, pm_ref, o_ref):
  s = _merge_s_tc(s_ref[...])
  o = jnp.where(s > -jnp.inf, p_ref[...] + s, 0.0)
  m = jnp.max(o, axis=0, keepdims=True)
  lse = jnp.log(jnp.sum(jnp.exp(o - m), axis=0, keepdims=True)) + m
  lsm = o - lse
  o_ref[...] = jnp.dot(pm_ref[...], lsm, preferred_element_type=jnp.float32).T


_PERM = np.concatenate([np.arange(0, _D, 2), np.arange(1, _D, 2)])
_PMAT = np.zeros((_D, _D), dtype=np.float32)
_PMAT[_PERM, np.arange(_D)] = 1.0


def kernel(x, edge_index, W1, b1, W2, b2):
  src = edge_index[0]
  dst = edge_index[1]
  perm = jnp.asarray(_PERM)
  w1p = W1[:, perm]
  b1c = b1[perm].reshape(_D, 1)
  w2p = W2[jnp.concatenate([perm, _D + perm])][:, perm]
  b2c = b2[perm].reshape(_D, 1)
  pmat = jnp.asarray(_PMAT)

  ft = jax.ShapeDtypeStruct((_D, _N), jnp.float32)
  qt = jax.ShapeDtypeStruct((_W, _N), jnp.int32)

  p1t, q1p = pl.pallas_call(
      _tc1_body,
      out_shape=[ft, qt],
  )(x, w1p, b1c)

  s1p = _sc_segmax(q1p, src, dst)

  p2t, q2p = pl.pallas_call(
      _tc2_body,
      out_shape=[ft, qt],
  )(p1t, s1p, w2p, b2c)

  s2p = _sc_segmax(q2p, src, dst)

  out = pl.pallas_call(
      _tc3_body,
      out_shape=jax.ShapeDtypeStruct((_N, _D), jnp.float32),
  )(p2t, s2p, pmat)

  return out


# speedup vs baseline: 1.3711x; 1.0363x over previous
import functools

import numpy as np

import jax
import jax.numpy as jnp
from jax import lax
from jax.experimental import pallas as pl
from jax.experimental.pallas import tpu as pltpu
from jax.experimental.pallas import tpu_sc as plsc

_N = 10000
_E = 320000
_D = 128
_W = _D // 2

_NC = 2
_NS = 16
_NW = _NC * _NS
_L = 16
_PPT = 4
_EH = _E // 2
_CHUNK = 6400
_GROUPS = _CHUNK // _L
_BLK = 4
_NBLKS = _GROUPS // _BLK
_NCHUNKS = _EH // _CHUNK

_NEGINF_PAIR = -8323200


def _unpack(w):
  lo = plsc.bitcast(jnp.left_shift(w, 16), jnp.float32)
  hi = plsc.bitcast(w & jnp.int32(-65536), jnp.float32)
  return lo, hi


def _pack(lo, hi):
  return plsc.bitcast(hi, jnp.int32) | lax.shift_right_logical(
      plsc.bitcast(lo, jnp.int32), 16)


def _sc_segmax(qp, src, dst):
  mesh = plsc.VectorSubcoreMesh(core_axis_name="c", subcore_axis_name="s")

  @functools.partial(
      pl.kernel,
      out_type=jax.ShapeDtypeStruct((2 * _W * _N,), jnp.int32),
      mesh=mesh,
      compiler_params=pltpu.CompilerParams(needs_layout_passes=False),
      scratch_types=[
          pltpu.VMEM((_PPT * _N,), jnp.int32),
          pltpu.VMEM((_PPT * _N,), jnp.int32),
          pltpu.VMEM((_CHUNK,), jnp.int32),
          pltpu.VMEM((_CHUNK,), jnp.int32),
          pltpu.VMEM((_CHUNK,), jnp.int32),
          pltpu.VMEM((_CHUNK,), jnp.int32),
          pltpu.SemaphoreType.DMA,
          pltpu.SemaphoreType.DMA,
          pltpu.SemaphoreType.DMA,
          pltpu.SemaphoreType.DMA,
      ],
  )
  def k(qp_hbm, src_hbm, dst_hbm, out_hbm, q_v, acc_v,
        sa_v, da_v, sb_v, db_v, sem_sa, sem_da, sem_sb, sem_db):
    wid = lax.axis_index("s") * _NC + lax.axis_index("c")
    half = wid // _NS
    slot = wid % _NS
    ebase = half * _EH
    pltpu.sync_copy(qp_hbm.at[pl.ds(slot * _PPT * _N, _PPT * _N)], q_v)

    negw = jnp.full((_L,), _NEGINF_PAIR, dtype=jnp.int32)
    poff = [jnp.full((_L,), p * _N, dtype=jnp.int32) for p in range(_PPT)]

    def init_row(i, _):
      acc_v[pl.ds(i * _L, _L)] = negw
      return 0

    lax.fori_loop(0, (_PPT * _N) // _L, init_row, 0)

    ones = jnp.ones((_L,), dtype=jnp.int32)
    lid = jnp.arange(_L, dtype=jnp.int32)
    idxp1 = jnp.minimum(lid + 1, _L - 1)
    lt15 = lid < (_L - 1)

    def slow_group(s_v, d_v, g):
      s16 = s_v[pl.ds(g * _L, _L)]
      d16 = d_v[pl.ds(g * _L, _L)]
      qw = [plsc.load_gather(q_v, [s16 + poff[p]]) for p in range(_PPT)]
      vals = [_unpack(w) for w in qw]
      didx = [d16 + poff[p] for p in range(_PPT)]

      def cond(pend):
        n = jnp.sum(pend[0])
        for p in range(1, _PPT):
          n = n + jnp.sum(pend[p])
        return n > 0

      def attempt(pend):
        for p in range(_PPT):
          cur = plsc.load_gather(acc_v, [didx[p]])
          clo, chi = _unpack(cur)
          neww = _pack(jnp.maximum(clo, vals[p][0]),
                       jnp.maximum(chi, vals[p][1]))
          plsc.store_scatter(acc_v, [didx[p]], neww, mask=pend[p] > 0)
        out = []
        for p in range(_PPT):
          chk = plsc.load_gather(acc_v, [didx[p]])
          klo, khi = _unpack(chk)
          bad = (klo < vals[p][0]) | (khi < vals[p][1])
          out.append(pend[p] & bad.astype(jnp.int32))
        return tuple(out)

      lax.while_loop(cond, attempt, (ones,) * _PPT)

    def block(s_v, d_v, bi, _):
      g0 = bi * _BLK
      fails = None
      for t in range(_BLK):
        g = g0 + t
        s16 = s_v[pl.ds(g * _L, _L)]
        d16 = d_v[pl.ds(g * _L, _L)]
        qw = [plsc.load_gather(q_v, [s16 + poff[p]]) for p in range(_PPT)]
        vals = [_unpack(w) for w in qw]
        didx = [d16 + poff[p] for p in range(_PPT)]
        ds = lax.sort(d16)
        nxt = lax.gather(
            ds, idxp1.reshape(_L, 1),
            lax.GatherDimensionNumbers(offset_dims=(), collapsed_slice_dims=(0,),
                                       start_index_map=(0,)),
            (1,), mode=lax.GatherScatterMode.PROMISE_IN_BOUNDS)
        f = ((ds == nxt) & lt15).astype(jnp.int32)
        fails = f if fails is None else (fails | f)
        for p in range(_PPT):
          cur = plsc.load_gather(acc_v, [didx[p]])
          clo, chi = _unpack(cur)
          neww = _pack(jnp.maximum(clo, vals[p][0]),
                       jnp.maximum(chi, vals[p][1]))
          plsc.store_scatter(acc_v, [didx[p]], neww)

      nfail = plsc.all_reduce_population_count(fails > 0)[0]

      @pl.when(nfail != 0)
      def _slow():
        for t in range(_BLK):
          slow_group(s_v, d_v, g0 + t)

      return 0

    def do_chunk(s_v, d_v):
      lax.fori_loop(0, _NBLKS, functools.partial(block, s_v, d_v), 0,
                    unroll=2)

    def start_copy(c, s_v, d_v, sem_s, sem_d):
      off = ebase + c * _CHUNK
      pltpu.async_copy(src_hbm.at[pl.ds(off, _CHUNK)], s_v, sem_s)
      pltpu.async_copy(dst_hbm.at[pl.ds(off, _CHUNK)], d_v, sem_d)

    def wait_copy(s_v, d_v, sem_s, sem_d):
      pltpu.make_async_copy(src_hbm.at[pl.ds(0, _CHUNK)], s_v, sem_s).wait()
      pltpu.make_async_copy(dst_hbm.at[pl.ds(0, _CHUNK)], d_v, sem_d).wait()

    start_copy(0, sa_v, da_v, sem_sa, sem_da)

    def chunk_pair(c2, _):
      c = c2 * 2
      start_copy(c + 1, sb_v, db_v, sem_sb, sem_db)
      wait_copy(sa_v, da_v, sem_sa, sem_da)
      do_chunk(sa_v, da_v)

      @pl.when(c2 + 1 < _NCHUNKS // 2)
      def _prefetch():
        start_copy(c + 2, sa_v, da_v, sem_sa, sem_da)

      wait_copy(sb_v, db_v, sem_sb, sem_db)
      do_chunk(sb_v, db_v)
      return 0

    lax.fori_loop(0, _NCHUNKS // 2, chunk_pair, 0)

    pltpu.sync_copy(
        acc_v, out_hbm.at[pl.ds((half * _W + slot * _PPT) * _N, _PPT * _N)])

  return k(qp.reshape(_W * _N), src, dst).reshape(2, _W, _N)


def _pack_q_tc(q):
  lo = lax.bitcast_convert_type(q[:_W].astype(jnp.bfloat16),
                                jnp.uint16).astype(jnp.uint32)
  hi = lax.bitcast_convert_type(q[_W:].astype(jnp.bfloat16),
                                jnp.uint16).astype(jnp.uint32)
  return lax.bitcast_convert_type((hi << 16) | lo, jnp.int32)


def _merge_s_tc(s2):
  out = None
  for h in range(2):
    w = s2[h]
    lo = lax.bitcast_convert_type(jnp.left_shift(w, 16), jnp.float32)
    hi = lax.bitcast_convert_type(w & jnp.int32(-65536), jnp.float32)
    s = jnp.concatenate([lo, hi], axis=0)
    out = s if out is None else jnp.maximum(out, s)
  return out


def _tc1_body(x_ref, w_ref, b_ref, p_ref, q_ref):
  x = x_ref[...]
  w = w_ref[...]
  wd = w[:_D] - w[_D:]
  ws = w[_D:]
  dn = (((0,), (1,)), ((), ()))
  p_ref[...] = lax.dot_general(wd, x, dn, preferred_element_type=jnp.float32) + b_ref[...]
  q_ref[...] = _pack_q_tc(
      lax.dot_general(ws, x, dn, preferred_element_type=jnp.float32))


def _tc2_body(p_ref, s_ref, w_ref, b_ref, p2_ref, q2_ref):
  s = _merge_s_tc(s_ref[...])
  h = jnp.where(s > -jnp.inf, p_ref[...] + s, 0.0)
  h = jnp.maximum(h, 0.0)
  w = w_ref[...]
  wd = w[:_D] - w[_D:]
  ws = w[_D:]
  dn = (((0,), (0,)), ((), ()))
  p2_ref[...] = lax.dot_general(wd, h, dn, preferred_element_type=jnp.float32) + b_ref[...]
  q2_ref[...] = _pack_q_tc(
      lax.dot_general(ws, h, dn, preferred_element_type=jnp.float32))


def _tc3_body(p_ref, s_ref, pm_ref, o_ref):
  s = _merge_s_tc(s_ref[...])
  o = jnp.where(s > -jnp.inf, p_ref[...] + s, 0.0)
  m = jnp.max(o, axis=0, keepdims=True)
  lse = jnp.log(jnp.sum(jnp.exp(o - m), axis=0, keepdims=True)) + m
  lsm = o - lse
  o_ref[...] = jnp.dot(pm_ref[...], lsm, preferred_element_type=jnp.float32).T


_PERM = np.concatenate([np.arange(0, _D, 2), np.arange(1, _D, 2)])
_PMAT = np.zeros((_D, _D), dtype=np.float32)
_PMAT[_PERM, np.arange(_D)] = 1.0


def kernel(x, edge_index, W1, b1, W2, b2):
  src = edge_index[0]
  dst = edge_index[1]
  perm = jnp.asarray(_PERM)
  w1p = W1[:, perm]
  b1c = b1[perm].reshape(_D, 1)
  w2p = W2[jnp.concatenate([perm, _D + perm])][:, perm]
  b2c = b2[perm].reshape(_D, 1)
  pmat = jnp.asarray(_PMAT)

  ft = jax.ShapeDtypeStruct((_D, _N), jnp.float32)
  qt = jax.ShapeDtypeStruct((_W, _N), jnp.int32)

  p1t, q1p = pl.pallas_call(
      _tc1_body,
      out_shape=[ft, qt],
  )(x, w1p, b1c)

  s1p = _sc_segmax(q1p, src, dst)

  p2t, q2p = pl.pallas_call(
      _tc2_body,
      out_shape=[ft, qt],
  )(p1t, s1p, w2p, b2c)

  s2p = _sc_segmax(q2p, src, dst)

  out = pl.pallas_call(
      _tc3_body,
      out_shape=jax.ShapeDtypeStruct((_N, _D), jnp.float32),
  )(p2t, s2p, pmat)

  return out
